# baseline (speedup 1.0000x reference)
import functools
import jax
import jax.numpy as jnp
from jax import lax
from jax.experimental import pallas as pl
from jax.experimental.pallas import tpu as pltpu
from jax.experimental.pallas import tpu_sc as plsc

N_NODES = 10000
N_EDGES = 5000
NNZ = 320000
D = 128

NC = 2
NS = 16
NW = NC * NS
C = 80
NNZ_PAD = NNZ
NNZ_PER_TILE = NNZ_PAD // NW
N_CHUNKS = NNZ_PER_TILE // C

NODE_PAD = 10240
EDGE_PAD = 5120

NSUP = 5
SUP = N_CHUNKS // NSUP
SUPC = SUP * C


def _make_spmm(n_seg_pad):
    seg_per_tile = n_seg_pad // NS
    n_zero_chunks = seg_per_tile // C
    mesh = plsc.VectorSubcoreMesh(core_axis_name="c", subcore_axis_name="s")

    @functools.partial(
        pl.kernel,
        out_type=jax.ShapeDtypeStruct((NC, n_seg_pad, D), jnp.float32),
        mesh=mesh,
        scratch_types=[
            pltpu.VMEM((SUPC,), jnp.int32),
            pltpu.VMEM((SUP, C), jnp.int32),
            pltpu.VMEM((SUPC,), jnp.float32),
            pltpu.VMEM((C, D), jnp.float32),
            pltpu.VMEM((C, D), jnp.float32),
            pltpu.VMEM_SHARED((n_seg_pad, D), jnp.float32),
            pltpu.SemaphoreType.DMA,
            pltpu.SemaphoreType.DMA,
            pltpu.SemaphoreType.DMA,
            pltpu.SemaphoreType.DMA,
        ],
    )
    def spmm(table_hbm, gidx_hbm, sidx_hbm, val_hbm, out_hbm,
             gidx_v, sidx_v, val_v, rows0_v, rows1_v, acc_sh,
             gsem0, gsem1, ssem0, ssem1):
        cid = lax.axis_index("c")
        sid = lax.axis_index("s")
        wid = cid * NS + sid

        def zrow(r, _):
            for j in range(D // 16):
                rows0_v[r, pl.ds(j * 16, 16)] = jnp.zeros((16,), jnp.float32)
            return 0
        lax.fori_loop(0, C, zrow, 0)

        def zacc(k, _):
            pltpu.sync_copy(rows0_v, acc_sh.at[pl.ds(sid * seg_per_tile + k * C, C)])
            return 0
        lax.fori_loop(0, n_zero_chunks, zacc, 0)
        plsc.subcore_barrier()

        def fire_g(k, rows_v, sem):
            pltpu.async_copy(table_hbm.at[gidx_v.at[pl.ds(k * C, C)]], rows_v, sem)

        def wait_g(k, rows_v, sem):
            pltpu.make_async_copy(
                table_hbm.at[gidx_v.at[pl.ds(k * C, C)]], rows_v, sem).wait()

        def fire_s(k, rows_v, sem):
            pltpu.async_copy(rows_v, acc_sh.at[sidx_v.at[k]], sem, add=True)

        def drain_s(rows_v, sem):
            pltpu.make_async_copy(table_hbm.at[pl.ds(0, C)], rows_v, sem).wait()

        def scale(k, rows_v):
            def scale16(g, _):
                row0 = g * 16
                vals = val_v[pl.ds(k * C + row0, 16)]
                for j in range(16):
                    bv = jnp.broadcast_to(vals[j], (16,))
                    for r in range(D // 16):
                        col = pl.ds(r * 16, 16)
                        rows_v[row0 + j, col] = rows_v[row0 + j, col] * bv
                return 0
            lax.fori_loop(0, C // 16, scale16, 0)

        def sup(s, _):
            base = wid * NNZ_PER_TILE + s * SUPC
            pltpu.sync_copy(gidx_hbm.at[pl.ds(base, SUPC)], gidx_v)
            pltpu.sync_copy(sidx_hbm.at[wid * NSUP + s], sidx_v)
            pltpu.sync_copy(val_hbm.at[pl.ds(base, SUPC)], val_v)

            fire_g(0, rows0_v, gsem0)
            fire_g(1, rows1_v, gsem1)

            def pair(i, _):
                k = i * 2
                wait_g(k, rows0_v, gsem0)
                scale(k, rows0_v)
                fire_s(k, rows0_v, ssem0)
                wait_g(k + 1, rows1_v, gsem1)
                scale(k + 1, rows1_v)
                fire_s(k + 1, rows1_v, ssem1)
                drain_s(rows0_v, ssem0)
                fire_g(k + 2, rows0_v, gsem0)
                drain_s(rows1_v, ssem1)

                @pl.when(k + 3 < SUP)
                def _():
                    fire_g(k + 3, rows1_v, gsem1)
                return 0
            lax.fori_loop(0, (SUP - 1) // 2, pair, 0)

            wait_g(SUP - 1, rows0_v, gsem0)
            scale(SUP - 1, rows0_v)
            fire_s(SUP - 1, rows0_v, ssem0)
            drain_s(rows0_v, ssem0)
            return 0
        lax.fori_loop(0, NSUP, sup, 0)

        plsc.subcore_barrier()

        def cout(k, _):
            off = sid * seg_per_tile + k * C
            pltpu.sync_copy(acc_sh.at[pl.ds(off, C)], rows0_v)
            pltpu.sync_copy(rows0_v, out_hbm.at[cid, pl.ds(off, C)])
            return 0
        lax.fori_loop(0, n_zero_chunks, cout, 0)

    return spmm


_spmm_edge = _make_spmm(EDGE_PAD)
_spmm_node = _make_spmm(NODE_PAD)


def _relu_bias_body(x_ref, b_ref, o_ref):
    o_ref[...] = jnp.maximum(x_ref[...], 0.0) + b_ref[...]


def _combine_body(p_ref, o_ref):
    o_ref[...] = p_ref[0] + p_ref[1]


def _softmax_mm_body(p_ref, w_ref, o_ref):
    s = p_ref[0] + p_ref[1]
    m = jnp.max(s, axis=1, keepdims=True)
    e = jnp.exp(s - m)
    sm = e / jnp.sum(e, axis=1, keepdims=True)
    o_ref[...] = jnp.dot(sm, w_ref[...], preferred_element_type=jnp.float32)


def kernel(x, inc_node, inc_edge, inc_val, hgc1_bias, fc1_W):
    h = pl.pallas_call(
        _relu_bias_body,
        grid=(10,),
        in_specs=[
            pl.BlockSpec((1000, D), lambda i: (i, 0)),
            pl.BlockSpec((1, D), lambda i: (0, 0)),
        ],
        out_specs=pl.BlockSpec((1000, D), lambda i: (i, 0)),
        out_shape=jax.ShapeDtypeStruct((N_NODES, D), jnp.float32),
    )(x, hgc1_bias.reshape(1, D))

    node_3d = inc_node.reshape(NW * NSUP, SUP, C)
    edge_3d = inc_edge.reshape(NW * NSUP, SUP, C)

    edge_p = _spmm_edge(h, inc_node, edge_3d, inc_val)

    edge = pl.pallas_call(
        _combine_body,
        grid=(5,),
        in_specs=[pl.BlockSpec((NC, 1000, D), lambda i: (0, i, 0))],
        out_specs=pl.BlockSpec((1000, D), lambda i: (i, 0)),
        out_shape=jax.ShapeDtypeStruct((N_EDGES, D), jnp.float32),
    )(edge_p)

    node_p = _spmm_node(edge, inc_edge, node_3d, inc_val)

    out = pl.pallas_call(
        _softmax_mm_body,
        grid=(10,),
        in_specs=[
            pl.BlockSpec((NC, 1000, D), lambda i: (0, i, 0)),
            pl.BlockSpec((D, D), lambda i: (0, 0)),
        ],
        out_specs=pl.BlockSpec((1000, D), lambda i: (i, 0)),
        out_shape=jax.ShapeDtypeStruct((N_NODES, D), jnp.float32),
    )(node_p, fc1_W.T)

    return (out, edge)


# speedup vs baseline: 1.0949x; 1.0949x over previous
import functools
import jax
import jax.numpy as jnp
from jax import lax
from jax.experimental import pallas as pl
from jax.experimental.pallas import tpu as pltpu
from jax.experimental.pallas import tpu_sc as plsc

N_NODES = 10000
N_EDGES = 5000
NNZ = 320000
D = 128

NC = 2
NS = 16
NW = NC * NS
C = 80
NNZ_PAD = NNZ
NNZ_PER_TILE = NNZ_PAD // NW
N_CHUNKS = NNZ_PER_TILE // C

NODE_PAD = 10240
EDGE_PAD = 5120

NSUP = 5
SUP = N_CHUNKS // NSUP
SUPC = SUP * C


def _make_spmm(n_seg_pad, with_act):
    seg_per_tile = n_seg_pad // NS
    n_zero_chunks = seg_per_tile // C
    mesh = plsc.VectorSubcoreMesh(core_axis_name="c", subcore_axis_name="s")

    @functools.partial(
        pl.kernel,
        out_type=jax.ShapeDtypeStruct((NC, n_seg_pad, D), jnp.float32),
        mesh=mesh,
        scratch_types=[
            pltpu.VMEM((SUPC,), jnp.int32),
            pltpu.VMEM((SUP, C), jnp.int32),
            pltpu.VMEM((SUPC,), jnp.float32),
            pltpu.VMEM((C, D), jnp.float32),
            pltpu.VMEM((C, D), jnp.float32),
            pltpu.VMEM((D,), jnp.float32),
            pltpu.VMEM_SHARED((n_seg_pad, D), jnp.float32),
            pltpu.SemaphoreType.DMA,
            pltpu.SemaphoreType.DMA,
        ],
    )
    def spmm(table_hbm, gidx_hbm, sidx_hbm, val_hbm, bias_hbm, out_hbm,
             gidx_v, sidx_v, val_v, rows0_v, rows1_v, bias_v, acc_sh,
             gsem0, gsem1):
        cid = lax.axis_index("c")
        sid = lax.axis_index("s")
        wid = cid * NS + sid

        pltpu.sync_copy(bias_hbm, bias_v)
        bias_regs = [bias_v[pl.ds(r * 16, 16)] for r in range(D // 16)]

        def zrow(r, _):
            for j in range(D // 16):
                rows0_v[r, pl.ds(j * 16, 16)] = jnp.zeros((16,), jnp.float32)
            return 0
        lax.fori_loop(0, C, zrow, 0)

        def zacc(k, _):
            pltpu.sync_copy(rows0_v, acc_sh.at[pl.ds(sid * seg_per_tile + k * C, C)])
            return 0
        lax.fori_loop(0, n_zero_chunks, zacc, 0)
        plsc.subcore_barrier()

        def fire_g(k, rows_v, sem):
            pltpu.async_copy(table_hbm.at[gidx_v.at[pl.ds(k * C, C)]], rows_v, sem)

        def wait_g(k, rows_v, sem):
            pltpu.make_async_copy(
                table_hbm.at[gidx_v.at[pl.ds(k * C, C)]], rows_v, sem).wait()

        def scale(k, rows_v):
            def scale16(g, _):
                row0 = g * 16
                vals = val_v[pl.ds(k * C + row0, 16)]
                for j in range(16):
                    bv = jnp.broadcast_to(vals[j], (16,))
                    for r in range(D // 16):
                        col = pl.ds(r * 16, 16)
                        t = rows_v[row0 + j, col]
                        if with_act:
                            t = jnp.maximum(t, 0.0) + bias_regs[r]
                        rows_v[row0 + j, col] = t * bv
                return 0
            lax.fori_loop(0, C // 16, scale16, 0)

        def sup(s, _):
            base = wid * NNZ_PER_TILE + s * SUPC
            pltpu.sync_copy(gidx_hbm.at[pl.ds(base, SUPC)], gidx_v)
            pltpu.sync_copy(sidx_hbm.at[wid * NSUP + s], sidx_v)
            pltpu.sync_copy(val_hbm.at[pl.ds(base, SUPC)], val_v)

            fire_g(0, rows0_v, gsem0)

            def process(k, rows_v, gsem):
                wait_g(k, rows_v, gsem)
                scale(k, rows_v)
                pltpu.sync_copy(rows_v, acc_sh.at[sidx_v.at[k]], add=True)

            def pair(i, _):
                k = i * 2
                fire_g(k + 1, rows1_v, gsem1)
                process(k, rows0_v, gsem0)
                fire_g(k + 2, rows0_v, gsem0)
                process(k + 1, rows1_v, gsem1)
                return 0
            lax.fori_loop(0, (SUP - 1) // 2, pair, 0)
            process(SUP - 1, rows0_v, gsem0)
            return 0
        lax.fori_loop(0, NSUP, sup, 0)

        plsc.subcore_barrier()

        def cout(k, _):
            off = sid * seg_per_tile + k * C
            pltpu.sync_copy(acc_sh.at[pl.ds(off, C)], rows0_v)
            pltpu.sync_copy(rows0_v, out_hbm.at[cid, pl.ds(off, C)])
            return 0
        lax.fori_loop(0, n_zero_chunks, cout, 0)

    return spmm


_spmm_edge = _make_spmm(EDGE_PAD, with_act=True)
_spmm_node = _make_spmm(NODE_PAD, with_act=False)


def _combine_body(p_ref, o_ref):
    o_ref[...] = p_ref[0] + p_ref[1]


def _softmax_mm_body(p_ref, w_ref, o_ref):
    s = p_ref[0] + p_ref[1]
    m = jnp.max(s, axis=1, keepdims=True)
    e = jnp.exp(s - m)
    sm = e / jnp.sum(e, axis=1, keepdims=True)
    o_ref[...] = jnp.dot(sm, w_ref[...], preferred_element_type=jnp.float32)


def kernel(x, inc_node, inc_edge, inc_val, hgc1_bias, fc1_W):
    node_3d = inc_node.reshape(NW * NSUP, SUP, C)
    edge_3d = inc_edge.reshape(NW * NSUP, SUP, C)

    edge_p = _spmm_edge(x, inc_node, edge_3d, inc_val, hgc1_bias)

    edge = pl.pallas_call(
        _combine_body,
        grid=(5,),
        in_specs=[pl.BlockSpec((NC, 1000, D), lambda i: (0, i, 0))],
        out_specs=pl.BlockSpec((1000, D), lambda i: (i, 0)),
        out_shape=jax.ShapeDtypeStruct((N_EDGES, D), jnp.float32),
    )(edge_p)

    node_p = _spmm_node(edge, inc_edge, node_3d, inc_val,
                        jnp.zeros((D,), jnp.float32))

    out = pl.pallas_call(
        _softmax_mm_body,
        grid=(10,),
        in_specs=[
            pl.BlockSpec((NC, 1000, D), lambda i: (0, i, 0)),
            pl.BlockSpec((D, D), lambda i: (0, 0)),
        ],
        out_specs=pl.BlockSpec((1000, D), lambda i: (i, 0)),
        out_shape=jax.ShapeDtypeStruct((N_NODES, D), jnp.float32),
    )(node_p, fc1_W.T)

    return (out, edge)


# speedup vs baseline: 1.0990x; 1.0038x over previous
import functools
import jax
import jax.numpy as jnp
from jax import lax
from jax.experimental import pallas as pl
from jax.experimental.pallas import tpu as pltpu
from jax.experimental.pallas import tpu_sc as plsc

N_NODES = 10000
N_EDGES = 5000
NNZ = 320000
D = 128

NC = 2
NS = 16
NW = NC * NS
C = 80
NNZ_PAD = NNZ
NNZ_PER_TILE = NNZ_PAD // NW
N_CHUNKS = NNZ_PER_TILE // C

NODE_PAD = 10240
EDGE_PAD = 5120

NSUP = 5
SUP = N_CHUNKS // NSUP
SUPC = SUP * C


def _make_spmm(n_seg_pad, with_act):
    seg_per_tile = n_seg_pad // NS
    n_zero_chunks = seg_per_tile // C
    mesh = plsc.VectorSubcoreMesh(core_axis_name="c", subcore_axis_name="s")

    @functools.partial(
        pl.kernel,
        out_type=jax.ShapeDtypeStruct((NC, n_seg_pad, D), jnp.float32),
        mesh=mesh,
        scratch_types=[
            pltpu.VMEM((SUPC,), jnp.int32),
            pltpu.VMEM((SUP, C), jnp.int32),
            pltpu.VMEM((SUPC,), jnp.float32),
            pltpu.VMEM((C, D), jnp.float32),
            pltpu.VMEM((C, D), jnp.float32),
            pltpu.VMEM((D,), jnp.float32),
            pltpu.VMEM_SHARED((n_seg_pad, D), jnp.float32),
            pltpu.SemaphoreType.DMA,
            pltpu.SemaphoreType.DMA,
        ],
    )
    def spmm(table_hbm, gidx_hbm, sidx_hbm, val_hbm, bias_hbm, out_hbm,
             gidx_v, sidx_v, val_v, rows0_v, rows1_v, bias_v, acc_sh,
             gsem0, gsem1):
        cid = lax.axis_index("c")
        sid = lax.axis_index("s")
        wid = cid * NS + sid

        pltpu.sync_copy(bias_hbm, bias_v)
        bias_regs = [bias_v[pl.ds(r * 16, 16)] for r in range(D // 16)]

        def zrow(r, _):
            for j in range(D // 16):
                rows0_v[r, pl.ds(j * 16, 16)] = jnp.zeros((16,), jnp.float32)
            return 0
        lax.fori_loop(0, C, zrow, 0)

        def zacc(k, _):
            pltpu.sync_copy(rows0_v, acc_sh.at[pl.ds(sid * seg_per_tile + k * C, C)])
            return 0
        lax.fori_loop(0, n_zero_chunks, zacc, 0)
        plsc.subcore_barrier()

        def fire_g(k, rows_v, sem):
            pltpu.async_copy(table_hbm.at[gidx_v.at[pl.ds(k * C, C)]], rows_v, sem)

        def wait_g(k, rows_v, sem):
            pltpu.make_async_copy(
                table_hbm.at[gidx_v.at[pl.ds(k * C, C)]], rows_v, sem).wait()

        def scale(k, rows_v):
            def scale16(g, _):
                row0 = g * 16
                vals = val_v[pl.ds(k * C + row0, 16)]
                for j in range(16):
                    bv = jnp.broadcast_to(vals[j], (16,))
                    for r in range(D // 16):
                        col = pl.ds(r * 16, 16)
                        t = rows_v[row0 + j, col]
                        if with_act:
                            t = jnp.maximum(t, 0.0) + bias_regs[r]
                        rows_v[row0 + j, col] = t * bv
                return 0
            lax.fori_loop(0, C // 16, scale16, 0)

        def sup(s, _):
            base = wid * NNZ_PER_TILE + s * SUPC
            pltpu.sync_copy(gidx_hbm.at[pl.ds(base, SUPC)], gidx_v)
            pltpu.sync_copy(sidx_hbm.at[wid * NSUP + s], sidx_v)
            pltpu.sync_copy(val_hbm.at[pl.ds(base, SUPC)], val_v)

            fire_g(0, rows0_v, gsem0)

            def process(k, rows_v, gsem):
                wait_g(k, rows_v, gsem)
                scale(k, rows_v)
                pltpu.sync_copy(rows_v, acc_sh.at[sidx_v.at[k]], add=True)

            def pair(i, _):
                k = i * 2
                fire_g(k + 1, rows1_v, gsem1)
                process(k, rows0_v, gsem0)
                fire_g(k + 2, rows0_v, gsem0)
                process(k + 1, rows1_v, gsem1)
                return 0
            lax.fori_loop(0, (SUP - 1) // 2, pair, 0)
            process(SUP - 1, rows0_v, gsem0)
            return 0
        lax.fori_loop(0, NSUP, sup, 0)

        plsc.subcore_barrier()

        off = sid * seg_per_tile
        pltpu.sync_copy(acc_sh.at[pl.ds(off, seg_per_tile)],
                        out_hbm.at[cid, pl.ds(off, seg_per_tile)])

    return spmm


_spmm_edge = _make_spmm(EDGE_PAD, with_act=True)
_spmm_node = _make_spmm(NODE_PAD, with_act=False)


def _combine_body(p_ref, o_ref):
    o_ref[...] = p_ref[0] + p_ref[1]


def _softmax_mm_body(p_ref, w_ref, o_ref):
    s = p_ref[0] + p_ref[1]
    m = jnp.max(s, axis=1, keepdims=True)
    e = jnp.exp(s - m)
    sm = e / jnp.sum(e, axis=1, keepdims=True)
    o_ref[...] = jnp.dot(sm, w_ref[...], preferred_element_type=jnp.float32)


def kernel(x, inc_node, inc_edge, inc_val, hgc1_bias, fc1_W):
    node_3d = inc_node.reshape(NW * NSUP, SUP, C)
    edge_3d = inc_edge.reshape(NW * NSUP, SUP, C)

    edge_p = _spmm_edge(x, inc_node, edge_3d, inc_val, hgc1_bias)

    edge = pl.pallas_call(
        _combine_body,
        grid=(5,),
        in_specs=[pl.BlockSpec((NC, 1000, D), lambda i: (0, i, 0))],
        out_specs=pl.BlockSpec((1000, D), lambda i: (i, 0)),
        out_shape=jax.ShapeDtypeStruct((N_EDGES, D), jnp.float32),
    )(edge_p)

    node_p = _spmm_node(edge, inc_edge, node_3d, inc_val,
                        jnp.zeros((D,), jnp.float32))

    out = pl.pallas_call(
        _softmax_mm_body,
        grid=(10,),
        in_specs=[
            pl.BlockSpec((NC, 1000, D), lambda i: (0, i, 0)),
            pl.BlockSpec((D, D), lambda i: (0, 0)),
        ],
        out_specs=pl.BlockSpec((1000, D), lambda i: (i, 0)),
        out_shape=jax.ShapeDtypeStruct((N_NODES, D), jnp.float32),
    )(node_p, fc1_W.T)

    return (out, edge)
